# baseline (speedup 1.0000x reference)
import functools

import jax
import jax.numpy as jnp
from jax import lax
from jax.experimental import pallas as pl
from jax.experimental.pallas import tpu as pltpu
from jax.experimental.pallas import tpu_sc as plsc

B = 2
SH, SW = 4, 8
OH, OW = 16, 224
N = OH * OW
K = 32
D1 = 128
D2 = 64
TQ = 128
TQT = 448

NCHUNK = 4
QC = B * N // NCHUNK

IDX_BITS = 12
IDX_MASK = (1 << IDX_BITS) - 1

SC_CORES = 2
SC_SUBCORES = 16
NW = SC_CORES * SC_SUBCORES
ROWS_PER_W = QC * K // NW
CH = 128
NCH = ROWS_PER_W // CH


def _sample(x):
    return x[:, ::SH, ::SW, :][:, :OH, :OW, :].reshape(x.shape[0], N, 3)


def _prep_body(p1_ref, p2_ref, We1_ref, be1_ref, We2_ref, be2_ref,
               Wcv1_ref, bcv1_ref, A_ref, C_ref):
    p1 = p1_ref[0]
    p2 = p2_ref[0]
    We1 = We1_ref[...]
    be1 = be1_ref[...]
    We2 = We2_ref[...]
    be2 = be2_ref[...]
    W1 = Wcv1_ref[...]
    b1 = bcv1_ref[...]

    def enc(p):
        h = jnp.maximum(jnp.dot(p, We1, preferred_element_type=jnp.float32) + be1, 0.0)
        return jnp.maximum(jnp.dot(h, We2, preferred_element_type=jnp.float32) + be2, 0.0)

    f1 = enc(p1)
    f2 = enc(p2)
    Wq = W1[0:64]
    Wk = W1[64:128]
    Wx = W1[128:131]
    A_ref[0] = (jnp.dot(f1, Wq, preferred_element_type=jnp.float32)
                - jnp.dot(p1, Wx, preferred_element_type=jnp.float32))
    C_ref[0] = (jnp.dot(f2, Wk, preferred_element_type=jnp.float32)
                + jnp.dot(p2, Wx, preferred_element_type=jnp.float32) + b1)


def _prep(p1, p2, We1, be1, We2, be2, Wcv1, bcv1):
    return pl.pallas_call(
        _prep_body,
        grid=(B,),
        in_specs=[
            pl.BlockSpec((1, N, 3), lambda b: (b, 0, 0)),
            pl.BlockSpec((1, N, 3), lambda b: (b, 0, 0)),
            pl.BlockSpec((3, 16), lambda b: (0, 0)),
            pl.BlockSpec((16,), lambda b: (0,)),
            pl.BlockSpec((16, 64), lambda b: (0, 0)),
            pl.BlockSpec((64,), lambda b: (0,)),
            pl.BlockSpec((131, 128), lambda b: (0, 0)),
            pl.BlockSpec((128,), lambda b: (0,)),
        ],
        out_specs=[
            pl.BlockSpec((1, N, D1), lambda b: (b, 0, 0)),
            pl.BlockSpec((1, N, D1), lambda b: (b, 0, 0)),
        ],
        out_shape=[
            jax.ShapeDtypeStruct((B, N, D1), jnp.float32),
            jax.ShapeDtypeStruct((B, N, D1), jnp.float32),
        ],
    )(p1, p2, We1, be1, We2, be2, Wcv1, bcv1)


def _topk_body(base, p1_ref, p2_ref, idx_ref, negd_ref):
    q = p1_ref[...]
    kp = p2_ref[...]
    qn = jnp.sum(q * q, axis=1, keepdims=True)
    kn = jnp.sum(kp * kp, axis=1)[None, :]
    qk = lax.dot_general(q, kp, (((1,), (1,)), ((), ())),
                         preferred_element_type=jnp.float32)
    d = qn - 2.0 * qk + kn
    bits = lax.bitcast_convert_type(jnp.maximum(d, 0.0), jnp.int32)
    iota = lax.broadcasted_iota(jnp.int32, (TQT, N), 1)
    key = (bits & ~IDX_MASK) | iota
    half = N // 2
    ka = key[:, :half]
    kb = key[:, half:]
    da = d[:, :half]
    db = d[:, half:]
    swap = kb < ka
    klo = jnp.where(swap, kb, ka)
    khi = jnp.where(swap, ka, kb)
    dlo = jnp.where(swap, db, da)
    dhi = jnp.where(swap, da, db)
    maxi = jnp.int32(0x7FFFFFFF)
    ones = jnp.ones((half, 1), jnp.float32)
    for k in range(K):
        mk = jnp.min(klo, axis=1, keepdims=True)
        idx_ref[:, k:k + 1] = (mk & IDX_MASK) + base
        eqm = klo == mk
        sel = jnp.where(eqm, dlo, 0.0)
        m = lax.dot_general(sel, ones, (((1,), (0,)), ((), ())),
                            preferred_element_type=jnp.float32)
        negd_ref[:, k:k + 1] = -m
        klo = jnp.where(eqm, khi, klo)
        dlo = jnp.where(eqm, dhi, dlo)
        khi = jnp.where(eqm, maxi, khi)


def _topk_part(p1c, p2b, base):
    return pl.pallas_call(
        functools.partial(_topk_body, base),
        grid=(QC // TQT,),
        in_specs=[
            pl.BlockSpec((TQT, 3), lambda t: (t, 0)),
            pl.BlockSpec((N, 3), lambda t: (0, 0)),
        ],
        out_specs=[pl.BlockSpec((TQT, K), lambda t: (t, 0)),
                   pl.BlockSpec((TQT, K), lambda t: (t, 0))],
        out_shape=[jax.ShapeDtypeStruct((QC, K), jnp.int32),
                   jax.ShapeDtypeStruct((QC, K), jnp.float32)],
    )(p1c, p2b)


def _sc_gather_part(C2, idxf):
    mesh = plsc.VectorSubcoreMesh(core_axis_name="c", subcore_axis_name="s",
                                  num_cores=SC_CORES, num_subcores=SC_SUBCORES)

    @functools.partial(
        pl.kernel,
        out_type=jax.ShapeDtypeStruct((QC * K, D1), jnp.float32),
        mesh=mesh,
        scratch_types=[
            pltpu.VMEM((CH,), jnp.int32),
            pltpu.VMEM((CH, D1), jnp.float32),
            pltpu.SemaphoreType.DMA,
        ],
    )
    def gather_kernel(C_hbm, idx_hbm, out_hbm, idx_v, rows_v, sem):
        wid = lax.axis_index("s") * SC_CORES + lax.axis_index("c")

        def body(i, carry):
            basei = wid * ROWS_PER_W + i * CH
            pltpu.sync_copy(idx_hbm.at[pl.ds(basei, CH)], idx_v)
            pltpu.async_copy(C_hbm.at[idx_v], rows_v, sem).wait()
            pltpu.sync_copy(rows_v, out_hbm.at[pl.ds(basei, CH)])
            return carry

        lax.fori_loop(0, NCH, body, 0)

    return gather_kernel(C2, idxf)


def _cv_body(A_ref, G_ref, negd_ref, W2_ref, b2_ref, W3_ref, b3_ref, out_ref):
    a = A_ref[...]
    g = G_ref[...]
    h1 = jnp.maximum(g.reshape(TQ, K, D1) + a[:, None, :], 0.0).reshape(TQ * K, D1)
    h2 = jnp.maximum(jnp.dot(h1, W2_ref[...], preferred_element_type=jnp.float32)
                     + b2_ref[...], 0.0)
    h3 = jnp.maximum(jnp.dot(h2, W3_ref[...], preferred_element_type=jnp.float32)
                     + b3_ref[...], 0.0)
    nd = negd_ref[...]
    mx = jnp.max(nd, axis=1, keepdims=True)
    e = jnp.exp(nd - mx)
    w = e / jnp.sum(e, axis=1, keepdims=True)
    out_ref[...] = jnp.sum(h3.reshape(TQ, K, D2) * w[:, :, None], axis=1)


def _cv_part(Ac, G, negdc, W2, b2, W3, b3):
    return pl.pallas_call(
        _cv_body,
        grid=(QC // TQ,),
        in_specs=[
            pl.BlockSpec((TQ, D1), lambda t: (t, 0)),
            pl.BlockSpec((TQ * K, D1), lambda t: (t, 0)),
            pl.BlockSpec((TQ, K), lambda t: (t, 0)),
            pl.BlockSpec((D1, D2), lambda t: (0, 0)),
            pl.BlockSpec((D2,), lambda t: (0,)),
            pl.BlockSpec((D2, D2), lambda t: (0, 0)),
            pl.BlockSpec((D2,), lambda t: (0,)),
        ],
        out_specs=pl.BlockSpec((TQ, D2), lambda t: (t, 0)),
        out_shape=jax.ShapeDtypeStruct((QC, D2), jnp.float32),
    )(Ac, G, negdc, W2, b2, W3, b3)


def kernel(xyz_f1, xyz_f2, W_enc1, b_enc1, W_enc2, b_enc2,
           W_cv1, b_cv1, W_cv2, b_cv2, W_cv3, b_cv3):
    p1 = _sample(xyz_f1)
    p2 = _sample(xyz_f2)
    A, C = _prep(p1, p2, W_enc1, b_enc1, W_enc2, b_enc2, W_cv1, b_cv1)
    C2 = C.reshape(B * N, D1)
    A2 = A.reshape(B * N, D1)
    p1f = p1.reshape(B * N, 3)
    outs = []
    for c in range(NCHUNK):
        b, qs = divmod(c * QC, N)
        idxc, negdc = _topk_part(p1f[c * QC:(c + 1) * QC], p2[b], b * N)
        Gc = _sc_gather_part(C2, idxc.reshape(QC * K))
        outs.append(_cv_part(A2[c * QC:(c + 1) * QC], Gc, negdc,
                             W_cv2, b_cv2, W_cv3, b_cv3))
    return jnp.concatenate(outs, axis=0).reshape(B, N, D2)


# speedup vs baseline: 1.2194x; 1.2194x over previous
import functools

import jax
import jax.numpy as jnp
from jax import lax
from jax.experimental import pallas as pl
from jax.experimental.pallas import tpu as pltpu
from jax.experimental.pallas import tpu_sc as plsc

B = 2
SH, SW = 4, 8
OH, OW = 16, 224
N = OH * OW
K = 32
D1 = 128
D2 = 64
TQ = 128
TQT = 128

NCHUNK = 4
QC = B * N // NCHUNK

IDX_BITS = 12
IDX_MASK = (1 << IDX_BITS) - 1

SC_CORES = 2
SC_SUBCORES = 16
NW = SC_CORES * SC_SUBCORES
ROWS_PER_W = QC * K // NW
CH = 128
NCH = ROWS_PER_W // CH


def _sample(x):
    return x[:, ::SH, ::SW, :][:, :OH, :OW, :].reshape(x.shape[0], N, 3)


def _prep_body(p1_ref, p2_ref, We1_ref, be1_ref, We2_ref, be2_ref,
               Wcv1_ref, bcv1_ref, A_ref, C_ref):
    p1 = p1_ref[0]
    p2 = p2_ref[0]
    We1 = We1_ref[...]
    be1 = be1_ref[...]
    We2 = We2_ref[...]
    be2 = be2_ref[...]
    W1 = Wcv1_ref[...]
    b1 = bcv1_ref[...]

    def enc(p):
        h = jnp.maximum(jnp.dot(p, We1, preferred_element_type=jnp.float32) + be1, 0.0)
        return jnp.maximum(jnp.dot(h, We2, preferred_element_type=jnp.float32) + be2, 0.0)

    f1 = enc(p1)
    f2 = enc(p2)
    Wq = W1[0:64]
    Wk = W1[64:128]
    Wx = W1[128:131]
    A_ref[0] = (jnp.dot(f1, Wq, preferred_element_type=jnp.float32)
                - jnp.dot(p1, Wx, preferred_element_type=jnp.float32))
    C_ref[0] = (jnp.dot(f2, Wk, preferred_element_type=jnp.float32)
                + jnp.dot(p2, Wx, preferred_element_type=jnp.float32) + b1)


def _prep(p1, p2, We1, be1, We2, be2, Wcv1, bcv1):
    return pl.pallas_call(
        _prep_body,
        grid=(B,),
        in_specs=[
            pl.BlockSpec((1, N, 3), lambda b: (b, 0, 0)),
            pl.BlockSpec((1, N, 3), lambda b: (b, 0, 0)),
            pl.BlockSpec((3, 16), lambda b: (0, 0)),
            pl.BlockSpec((16,), lambda b: (0,)),
            pl.BlockSpec((16, 64), lambda b: (0, 0)),
            pl.BlockSpec((64,), lambda b: (0,)),
            pl.BlockSpec((131, 128), lambda b: (0, 0)),
            pl.BlockSpec((128,), lambda b: (0,)),
        ],
        out_specs=[
            pl.BlockSpec((1, N, D1), lambda b: (b, 0, 0)),
            pl.BlockSpec((1, N, D1), lambda b: (b, 0, 0)),
        ],
        out_shape=[
            jax.ShapeDtypeStruct((B, N, D1), jnp.float32),
            jax.ShapeDtypeStruct((B, N, D1), jnp.float32),
        ],
    )(p1, p2, We1, be1, We2, be2, Wcv1, bcv1)


def _topk_body(base, p1_ref, p2_ref, idx_ref, negd_ref):
    q = p1_ref[...]
    kp = p2_ref[...]
    qn = jnp.sum(q * q, axis=1, keepdims=True)
    kn = jnp.sum(kp * kp, axis=1)[None, :]
    qk = lax.dot_general(q, kp, (((1,), (1,)), ((), ())),
                         preferred_element_type=jnp.float32)
    d = qn - 2.0 * qk + kn
    bits = lax.bitcast_convert_type(jnp.maximum(d, 0.0), jnp.int32)
    iota = lax.broadcasted_iota(jnp.int32, (TQT, N), 1)
    key = (bits & ~IDX_MASK) | iota
    half = N // 2
    ka = key[:, :half]
    kb = key[:, half:]
    da = d[:, :half]
    db = d[:, half:]
    swap = kb < ka
    klo = jnp.where(swap, kb, ka)
    khi = jnp.where(swap, ka, kb)
    dlo = jnp.where(swap, db, da)
    dhi = jnp.where(swap, da, db)
    maxi = jnp.int32(0x7FFFFFFF)
    ones = jnp.ones((half, 1), jnp.float32)
    for k in range(K):
        mk = jnp.min(klo, axis=1, keepdims=True)
        idx_ref[:, k:k + 1] = (mk & IDX_MASK) + base
        eqm = klo == mk
        sel = jnp.where(eqm, dlo, 0.0)
        m = lax.dot_general(sel, ones, (((1,), (0,)), ((), ())),
                            preferred_element_type=jnp.float32)
        negd_ref[:, k:k + 1] = -m
        klo = jnp.where(eqm, khi, klo)
        dlo = jnp.where(eqm, dhi, dlo)
        khi = jnp.where(eqm, maxi, khi)


def _topk_part(p1c, p2b, base):
    return pl.pallas_call(
        functools.partial(_topk_body, base),
        grid=(QC // TQT,),
        in_specs=[
            pl.BlockSpec((TQT, 3), lambda t: (t, 0)),
            pl.BlockSpec((N, 3), lambda t: (0, 0)),
        ],
        out_specs=[pl.BlockSpec((TQT, K), lambda t: (t, 0)),
                   pl.BlockSpec((TQT, K), lambda t: (t, 0))],
        out_shape=[jax.ShapeDtypeStruct((QC, K), jnp.int32),
                   jax.ShapeDtypeStruct((QC, K), jnp.float32)],
    )(p1c, p2b)


def _sc_gather_part(C2, idxf):
    mesh = plsc.VectorSubcoreMesh(core_axis_name="c", subcore_axis_name="s",
                                  num_cores=SC_CORES, num_subcores=SC_SUBCORES)

    @functools.partial(
        pl.kernel,
        out_type=jax.ShapeDtypeStruct((QC * K, D1), jnp.float32),
        mesh=mesh,
        scratch_types=[
            pltpu.VMEM((CH,), jnp.int32),
            pltpu.VMEM((CH, D1), jnp.float32),
            pltpu.SemaphoreType.DMA,
        ],
    )
    def gather_kernel(C_hbm, idx_hbm, out_hbm, idx_v, rows_v, sem):
        wid = lax.axis_index("s") * SC_CORES + lax.axis_index("c")

        def body(i, carry):
            basei = wid * ROWS_PER_W + i * CH
            pltpu.sync_copy(idx_hbm.at[pl.ds(basei, CH)], idx_v)
            pltpu.async_copy(C_hbm.at[idx_v], rows_v, sem).wait()
            pltpu.sync_copy(rows_v, out_hbm.at[pl.ds(basei, CH)])
            return carry

        lax.fori_loop(0, NCH, body, 0)

    return gather_kernel(C2, idxf)


def _cv_body(A_ref, G_ref, negd_ref, W2_ref, b2_ref, W3_ref, b3_ref, out_ref):
    a = A_ref[...]
    g = G_ref[...]
    h1 = jnp.maximum(g.reshape(TQ, K, D1) + a[:, None, :], 0.0).reshape(TQ * K, D1)
    h2 = jnp.maximum(jnp.dot(h1, W2_ref[...], preferred_element_type=jnp.float32)
                     + b2_ref[...], 0.0)
    h3 = jnp.maximum(jnp.dot(h2, W3_ref[...], preferred_element_type=jnp.float32)
                     + b3_ref[...], 0.0)
    nd = negd_ref[...]
    mx = jnp.max(nd, axis=1, keepdims=True)
    e = jnp.exp(nd - mx)
    w = e / jnp.sum(e, axis=1, keepdims=True)
    out_ref[...] = jnp.sum(h3.reshape(TQ, K, D2) * w[:, :, None], axis=1)


def _cv_part(Ac, G, negdc, W2, b2, W3, b3):
    return pl.pallas_call(
        _cv_body,
        grid=(QC // TQ,),
        in_specs=[
            pl.BlockSpec((TQ, D1), lambda t: (t, 0)),
            pl.BlockSpec((TQ * K, D1), lambda t: (t, 0)),
            pl.BlockSpec((TQ, K), lambda t: (t, 0)),
            pl.BlockSpec((D1, D2), lambda t: (0, 0)),
            pl.BlockSpec((D2,), lambda t: (0,)),
            pl.BlockSpec((D2, D2), lambda t: (0, 0)),
            pl.BlockSpec((D2,), lambda t: (0,)),
        ],
        out_specs=pl.BlockSpec((TQ, D2), lambda t: (t, 0)),
        out_shape=jax.ShapeDtypeStruct((QC, D2), jnp.float32),
    )(Ac, G, negdc, W2, b2, W3, b3)


def kernel(xyz_f1, xyz_f2, W_enc1, b_enc1, W_enc2, b_enc2,
           W_cv1, b_cv1, W_cv2, b_cv2, W_cv3, b_cv3):
    p1 = _sample(xyz_f1)
    p2 = _sample(xyz_f2)
    A, C = _prep(p1, p2, W_enc1, b_enc1, W_enc2, b_enc2, W_cv1, b_cv1)
    C2 = C.reshape(B * N, D1)
    A2 = A.reshape(B * N, D1)
    p1f = p1.reshape(B * N, 3)
    outs = []
    for c in range(NCHUNK):
        b, qs = divmod(c * QC, N)
        idxc, negdc = _topk_part(p1f[c * QC:(c + 1) * QC], p2[b], b * N)
        Gc = _sc_gather_part(C2, idxc.reshape(QC * K))
        outs.append(_cv_part(A2[c * QC:(c + 1) * QC], Gc, negdc,
                             W_cv2, b_cv2, W_cv3, b_cv3))
    return jnp.concatenate(outs, axis=0).reshape(B, N, D2)
